# baseline (speedup 1.0000x reference)
import functools

import jax
import jax.numpy as jnp
from jax import lax
from jax.experimental import pallas as pl
from jax.experimental.pallas import tpu as pltpu
from jax.experimental.pallas import tpu_sc as plsc

D_MODEL = 64
SCALE = 8.0
NUM_WORKERS = 32
CHUNK = 128
LANES = 16
HBUF = 2


def _relayout_scale(lut_t):
    d, v = lut_t.shape
    blk = 512

    def body(in_ref, out_ref):
        t = jnp.transpose(in_ref[...]) * SCALE
        t2 = t.reshape(blk // 2, 2, d)
        out_ref[...] = jnp.concatenate([t2[:, 0, :], t2[:, 1, :]], axis=1)

    return pl.pallas_call(
        body,
        grid=(v // blk,),
        in_specs=[pl.BlockSpec((d, blk), lambda i: (0, i))],
        out_specs=pl.BlockSpec((blk // 2, 2 * d), lambda i: (i, 0)),
        out_shape=jax.ShapeDtypeStruct((v // 2, 2 * d), jnp.float32),
    )(lut_t)


def _gather_scale_t(idx_t_arr, lut, batch, hist):
    b_per_w = batch // NUM_WORKERS
    bchunks = b_per_w // CHUNK

    mesh = plsc.VectorSubcoreMesh(core_axis_name="c", subcore_axis_name="s")

    @functools.partial(
        pl.kernel,
        mesh=mesh,
        out_type=jax.ShapeDtypeStruct(
            (hist, D_MODEL // 8, batch // CHUNK, 8, CHUNK), jnp.float32
        ),
        scratch_types=[
            pltpu.VMEM((hist, b_per_w), jnp.int32),
            pltpu.VMEM((bchunks, CHUNK, D_MODEL), jnp.float32),
            pltpu.VMEM((HBUF, D_MODEL // 8, 8, b_per_w + 1), jnp.float32),
            [pltpu.SemaphoreType.DMA] * bchunks,
            [pltpu.SemaphoreType.DMA] * HBUF,
        ],
        compiler_params=pltpu.CompilerParams(
            use_tc_tiling_on_sc=False, needs_layout_passes=False
        ),
    )
    def k(lut_hbm, idx_hbm, out_hbm, idx_t, gbufs, hbufs, gsems, wsems):
        wid = lax.axis_index("s") * 2 + lax.axis_index("c")
        b0 = wid * b_per_w
        bt0 = wid * bchunks
        pltpu.sync_copy(idx_hbm.at[:, pl.ds(b0, b_per_w)], idx_t)

        iota = lax.iota(jnp.int32, LANES)

        def start_gather(h, bb):
            pltpu.async_copy(
                lut_hbm.at[idx_t.at[h, pl.ds(bb * CHUNK, CHUNK)]],
                gbufs.at[bb],
                gsems[bb],
            )

        for bb in range(bchunks):
            start_gather(0, bb)

        def outer_body(h2, carry):
            for hh in range(HBUF):
                h = h2 * HBUF + hh
                hb = hbufs.at[hh]

                @pl.when(h2 > 0)
                def _():
                    for i in range(bchunks):
                        pltpu.make_async_copy(
                            hbufs.at[hh, :, :, pl.ds(0, CHUNK)],
                            out_hbm.at[0, :, 0, :, :],
                            wsems[hh],
                        ).wait()

                for bb in range(bchunks):
                    gb = gbufs.at[bb]
                    pltpu.make_async_copy(
                        lut_hbm.at[idx_t.at[0, pl.ds(0, CHUNK)]],
                        gb,
                        gsems[bb],
                    ).wait()

                    @plsc.parallel_loop(0, CHUNK, step=1, unroll=4)
                    def tr_r(r):
                        cols = jnp.full((LANES,), bb * CHUNK + r, jnp.int32)
                        for j in range(D_MODEL // LANES):
                            rows = iota + j * LANES
                            v = gb[r, pl.ds(j * LANES, LANES)]
                            plsc.store_scatter(
                                hb, [rows >> 3, rows & 7, cols], v
                            )

                    @pl.when(h < hist - 1)
                    def _():
                        start_gather(h + 1, bb)

                for i in range(bchunks):
                    pltpu.async_copy(
                        hbufs.at[hh, :, :, pl.ds(i * CHUNK, CHUNK)],
                        out_hbm.at[h, :, bt0 + i, :, :],
                        wsems[hh],
                    )
            return carry

        lax.fori_loop(0, hist // HBUF, outer_body, 0)

        for hh in range(HBUF):
            for i in range(bchunks):
                pltpu.make_async_copy(
                    hbufs.at[hh, :, :, pl.ds(0, CHUNK)],
                    out_hbm.at[0, :, 0, :, :],
                    wsems[hh],
                ).wait()

    return k(lut, idx_t_arr)


def kernel(x, lut):
    batch, hist = x.shape
    xt = jnp.transpose(x)
    lut_rm = _relayout_scale(jnp.transpose(lut)).reshape(lut.shape)
    out5 = _gather_scale_t(xt, lut_rm, batch, hist)
    return jnp.transpose(out5, (2, 4, 0, 1, 3)).reshape(batch, hist, D_MODEL)


# speedup vs baseline: 1.8716x; 1.8716x over previous
import functools

import jax
import jax.numpy as jnp
from jax import lax
from jax.experimental import pallas as pl
from jax.experimental.pallas import tpu as pltpu
from jax.experimental.pallas import tpu_sc as plsc

D_MODEL = 64
SCALE = 8.0
NUM_WORKERS = 32
CHUNK = 128
LANES = 16
HBUF = 2


def _gather_scale_t(idx_t_arr, lut, batch, hist):
    b_per_w = batch // NUM_WORKERS
    bchunks = b_per_w // CHUNK

    mesh = plsc.VectorSubcoreMesh(core_axis_name="c", subcore_axis_name="s")

    @functools.partial(
        pl.kernel,
        mesh=mesh,
        out_type=jax.ShapeDtypeStruct(
            (hist, D_MODEL // 8, batch // CHUNK, 8, CHUNK), jnp.float32
        ),
        scratch_types=[
            pltpu.VMEM((hist, b_per_w), jnp.int32),
            pltpu.VMEM((bchunks, CHUNK, D_MODEL), jnp.float32),
            pltpu.VMEM((HBUF, D_MODEL // 8, 8, b_per_w + 1), jnp.float32),
            [pltpu.SemaphoreType.DMA] * bchunks,
            [pltpu.SemaphoreType.DMA] * HBUF,
        ],
        compiler_params=pltpu.CompilerParams(
            use_tc_tiling_on_sc=False, needs_layout_passes=False
        ),
    )
    def k(lut_hbm, idx_hbm, out_hbm, idx_t, gbufs, hbufs, gsems, wsems):
        wid = lax.axis_index("s") * 2 + lax.axis_index("c")
        b0 = wid * b_per_w
        bt0 = wid * bchunks
        pltpu.sync_copy(idx_hbm.at[:, pl.ds(b0, b_per_w)], idx_t)

        iota = lax.iota(jnp.int32, LANES)

        def start_gather(h, bb):
            pltpu.async_copy(
                lut_hbm.at[idx_t.at[h, pl.ds(bb * CHUNK, CHUNK)]],
                gbufs.at[bb],
                gsems[bb],
            )

        for bb in range(bchunks):
            start_gather(0, bb)

        def outer_body(h2, carry):
            for hh in range(HBUF):
                h = h2 * HBUF + hh
                hb = hbufs.at[hh]

                @pl.when(h2 > 0)
                def _():
                    for i in range(bchunks):
                        pltpu.make_async_copy(
                            hbufs.at[hh, :, :, pl.ds(0, CHUNK)],
                            out_hbm.at[0, :, 0, :, :],
                            wsems[hh],
                        ).wait()

                for bb in range(bchunks):
                    gb = gbufs.at[bb]
                    pltpu.make_async_copy(
                        lut_hbm.at[idx_t.at[0, pl.ds(0, CHUNK)]],
                        gb,
                        gsems[bb],
                    ).wait()

                    @plsc.parallel_loop(0, CHUNK, step=1, unroll=4)
                    def tr_r(r):
                        cols = jnp.full((LANES,), bb * CHUNK + r, jnp.int32)
                        for j in range(D_MODEL // LANES):
                            rows = iota + j * LANES
                            v = gb[r, pl.ds(j * LANES, LANES)] * SCALE
                            plsc.store_scatter(
                                hb, [rows >> 3, rows & 7, cols], v
                            )

                    @pl.when(h < hist - 1)
                    def _():
                        start_gather(h + 1, bb)

                for i in range(bchunks):
                    pltpu.async_copy(
                        hbufs.at[hh, :, :, pl.ds(i * CHUNK, CHUNK)],
                        out_hbm.at[h, :, bt0 + i, :, :],
                        wsems[hh],
                    )
            return carry

        lax.fori_loop(0, hist // HBUF, outer_body, 0)

        for hh in range(HBUF):
            for i in range(bchunks):
                pltpu.make_async_copy(
                    hbufs.at[hh, :, :, pl.ds(0, CHUNK)],
                    out_hbm.at[0, :, 0, :, :],
                    wsems[hh],
                ).wait()

    return k(lut, idx_t_arr)


def kernel(x, lut):
    batch, hist = x.shape
    xt = jnp.transpose(x)
    out5 = _gather_scale_t(xt, lut, batch, hist)
    return jnp.transpose(out5, (2, 4, 0, 1, 3)).reshape(batch, hist, D_MODEL)
